# Initial kernel scaffold; baseline (speedup 1.0000x reference)
import jax
import jax.numpy as jnp
from jax.experimental import pallas as pl


def kernel(u, x, tar, neg, offset, isEval, EUI, EIU, EIL, ELI):
    raise NotImplementedError("write your pallas kernel here")


# speedup vs baseline: 4.7355x; 4.7355x over previous
import functools

import jax
import jax.numpy as jnp
from jax import lax
from jax.experimental import pallas as pl
from jax.experimental.pallas import tpu as pltpu
from jax.experimental.pallas import tpu_sc as plsc

B = 4096
L = 50
T = 10
D = 64

NC = 2
NS = 16
NW = NC * NS
BPT = B // NW
SB = 8
NSTEP = BPT // SB
XCOLS = 100
XG = SB * L // XCOLS
PPS = SB * T
NGRP = PPS // 16


def _sc_body(u_hbm, x_hbm, tar_hbm, neg_hbm, off_hbm, eui_t, eiu_t, eil_t,
             eli_t, outT_hbm, outN_hbm,
             u_v, x_v, tar_v, neg_v, off_v, eui_v,
             eli_rows, tarU_rows, tarL_rows, negU_rows, negL_rows,
             trT_v, trN_v, scT_v, scN_v, gsem, esem):
    wid = lax.axis_index("s") * NC + lax.axis_index("c")
    base = wid * BPT

    pltpu.sync_copy(u_hbm.at[pl.ds(base, BPT)], u_v)
    pltpu.sync_copy(x_hbm.at[pl.ds(wid * (BPT * L // XCOLS), BPT * L // XCOLS)], x_v)
    pltpu.sync_copy(tar_hbm.at[pl.ds(wid * NSTEP, NSTEP)], tar_v)
    pltpu.sync_copy(neg_hbm.at[pl.ds(wid * NSTEP, NSTEP)], neg_v)
    pltpu.sync_copy(off_hbm.at[pl.ds(base, BPT)], off_v)

    pltpu.async_copy(eui_t.at[u_v], eui_v, esem).wait()

    iot16 = lax.iota(jnp.int32, 16)

    def step(j, _):
        cps = []
        for g in range(XG):
            cps.append(pltpu.async_copy(
                eli_t.at[x_v.at[j * XG + g]],
                eli_rows.at[pl.ds(g * XCOLS, XCOLS)], gsem))
        cps.append(pltpu.async_copy(eiu_t.at[tar_v.at[j]], tarU_rows, gsem))
        cps.append(pltpu.async_copy(eil_t.at[tar_v.at[j]], tarL_rows, gsem))
        cps.append(pltpu.async_copy(eiu_t.at[neg_v.at[j]], negU_rows, gsem))
        cps.append(pltpu.async_copy(eil_t.at[neg_v.at[j]], negL_rows, gsem))
        for c in cps:
            c.wait()

        for b in range(SB):
            bg = j * SB + b
            a = [eui_v[bg, pl.ds(16 * k, 16)] for k in range(4)]

            def lbody(l, e):
                r = b * L + l
                return tuple(e[k] + eli_rows[r, pl.ds(16 * k, 16)]
                             for k in range(4))

            e = lax.fori_loop(
                0, L, lbody,
                tuple(jnp.zeros((16,), jnp.float32) for _ in range(4)))
            off_s = plsc.load_gather(
                off_v, [jnp.full((16,), bg, jnp.int32)]) * (1.0 / L)
            e = [ek * off_s for ek in e]

            for t in range(T):
                r = b * T + t
                accT = a[0] * tarU_rows[r, pl.ds(0, 16)]
                accN = a[0] * negU_rows[r, pl.ds(0, 16)]
                for k in range(1, 4):
                    accT = accT + a[k] * tarU_rows[r, pl.ds(16 * k, 16)]
                    accN = accN + a[k] * negU_rows[r, pl.ds(16 * k, 16)]
                for k in range(4):
                    accT = accT + e[k] * tarL_rows[r, pl.ds(16 * k, 16)]
                    accN = accN + e[k] * negL_rows[r, pl.ds(16 * k, 16)]
                trT_v[pl.ds(r * 16, 16)] = accT
                trN_v[pl.ds(r * 16, 16)] = accN

        for g in range(NGRP):
            sT = jnp.zeros((16,), jnp.float32)
            sN = jnp.zeros((16,), jnp.float32)
            for c in range(16):
                idx = iot16 * 16 + (g * 256 + c)
                sT = sT + plsc.load_gather(trT_v, [idx])
                sN = sN + plsc.load_gather(trN_v, [idx])
            scT_v[pl.ds(j * PPS + g * 16, 16)] = sT
            scN_v[pl.ds(j * PPS + g * 16, 16)] = sN
        return _

    lax.fori_loop(0, NSTEP, step, None)

    pltpu.sync_copy(scT_v, outT_hbm.at[pl.ds(base * T, BPT * T)])
    pltpu.sync_copy(scN_v, outN_hbm.at[pl.ds(base * T, BPT * T)])


@functools.partial(
    pl.kernel,
    out_type=(jax.ShapeDtypeStruct((B * T,), jnp.float32),
              jax.ShapeDtypeStruct((B * T,), jnp.float32)),
    mesh=plsc.VectorSubcoreMesh(core_axis_name="c", subcore_axis_name="s"),
    scratch_types=[
        pltpu.VMEM((BPT,), jnp.int32),
        pltpu.VMEM((B * L // NW // XCOLS, XCOLS), jnp.int32),
        pltpu.VMEM((NSTEP, PPS), jnp.int32),
        pltpu.VMEM((NSTEP, PPS), jnp.int32),
        pltpu.VMEM((BPT,), jnp.float32),
        pltpu.VMEM((BPT, D), jnp.float32),
        pltpu.VMEM((SB * L, D), jnp.float32),
        pltpu.VMEM((PPS, D), jnp.float32),
        pltpu.VMEM((PPS, D), jnp.float32),
        pltpu.VMEM((PPS, D), jnp.float32),
        pltpu.VMEM((PPS, D), jnp.float32),
        pltpu.VMEM((PPS * 16,), jnp.float32),
        pltpu.VMEM((PPS * 16,), jnp.float32),
        pltpu.VMEM((BPT * T,), jnp.float32),
        pltpu.VMEM((BPT * T,), jnp.float32),
        pltpu.SemaphoreType.DMA,
        pltpu.SemaphoreType.DMA,
    ],
    compiler_params=pltpu.CompilerParams(needs_layout_passes=False,
                                         use_tc_tiling_on_sc=False),
)
def _fpmc_sc(u, x2, tar2, neg2, off, EUI, EIU, EIL, ELI, outT, outN, *scratch):
    _sc_body(u, x2, tar2, neg2, off, EUI, EIU, EIL, ELI, outT, outN, *scratch)


def kernel(u, x, tar, neg, offset, isEval, EUI, EIU, EIL, ELI):
    x2 = x.reshape(B * L // XCOLS, XCOLS)
    tar2 = tar.reshape(B * T // PPS, PPS)
    neg2 = neg.reshape(B * T // PPS, PPS)
    off = offset.reshape(B)
    sT, sN = _fpmc_sc(u, x2, tar2, neg2, off, EUI, EIU, EIL, ELI)
    sT = sT.reshape(B, T)
    sN = sN.reshape(B, T)
    second = jnp.where(jnp.asarray(isEval), jnp.zeros_like(sN), sN)
    return (sT, second)
